# baseline (speedup 1.0000x reference)
import functools

import jax
import jax.numpy as jnp
from jax import lax
from jax.experimental import pallas as pl
from jax.experimental.pallas import tpu as pltpu
from jax.experimental.pallas import tpu_sc as plsc

N_NODES = 10000
N_EDGES = 160000
N_GRAPHS = 64

NS = 16
FC = 64
EPT = N_EDGES // NS
BATCH = 100
NB = EPT // BATCH
ZROWS = 104
RPT = 624

_MESH = lambda: plsc.VectorSubcoreMesh(core_axis_name="c", subcore_axis_name="s")


def _zero_vmem(ref, nrows, ncols):
    zero16 = jnp.zeros((16,), jnp.float32)

    def zr(i, _):
        def zc(j, _):
            ref[i, pl.ds(j * 16, 16)] = zero16
            return 0
        return lax.fori_loop(0, ncols // 16, zc, 0)

    lax.fori_loop(0, nrows, zr, 0)


def _deg_body(dstr, out, didx, ones_v, zrow, acc):
    tid = lax.axis_index("s")
    core = lax.axis_index("c")

    @pl.when(core == 0)
    def _():
        pltpu.sync_copy(dstr.at[tid], didx)
        onerow = jnp.where(lax.iota(jnp.int32, 16) == 0, 1.0, 0.0).astype(jnp.float32)

        def fill(i, _):
            ones_v[i, :] = onerow
            return 0
        lax.fori_loop(0, BATCH, fill, 0)
        _zero_vmem(zrow, ZROWS, 16)
        for j in range(RPT // ZROWS):
            pltpu.sync_copy(zrow, acc.at[pl.ds(tid * RPT + j * ZROWS, ZROWS)])

        @pl.when(tid == NS - 1)
        def _():
            pltpu.sync_copy(zrow.at[pl.ds(0, 16)],
                            acc.at[pl.ds(NS * RPT, N_NODES - NS * RPT)])
        plsc.subcore_barrier()

        def body(b, _):
            pltpu.sync_copy(ones_v, acc.at[didx.at[b]], add=True)
            return 0
        lax.fori_loop(0, NB, body, 0)
        plsc.subcore_barrier()
        pltpu.sync_copy(acc.at[pl.ds(tid * RPT, RPT)],
                        out.at[pl.ds(tid * RPT, RPT)])

        @pl.when(tid == NS - 1)
        def _():
            pltpu.sync_copy(acc.at[pl.ds(NS * RPT, N_NODES - NS * RPT)],
                            out.at[pl.ds(NS * RPT, N_NODES - NS * RPT)])


def _sc_degree(dst_r):
    return pl.kernel(
        _deg_body,
        mesh=_MESH(),
        compiler_params=pltpu.CompilerParams(use_tc_tiling_on_sc=False),
        out_type=jax.ShapeDtypeStruct((N_NODES, 16), jnp.float32),
        scratch_types=[
            pltpu.VMEM((NB, BATCH), jnp.int32),
            pltpu.VMEM((BATCH, 16), jnp.float32),
            pltpu.VMEM((ZROWS, 16), jnp.float32),
            pltpu.VMEM_SHARED((N_NODES, 16), jnp.float32),
        ],
    )(dst_r)


_NBUF = 8
_LOOK = 4


def _agg_body(nch, srcr, dstr, *rest):
    ins = rest[:nch]
    outs = rest[nch:2 * nch]
    sidx = rest[2 * nch]
    didx = rest[2 * nch + 1]
    bufs = rest[2 * nch + 2:2 * nch + 2 + _NBUF]
    zrow = rest[2 * nch + 2 + _NBUF]
    acc = rest[2 * nch + 3 + _NBUF]
    gsems = rest[2 * nch + 4 + _NBUF:2 * nch + 4 + 2 * _NBUF]
    ssems = rest[2 * nch + 4 + 2 * _NBUF:]
    tid = lax.axis_index("s")
    core = lax.axis_index("c")

    pltpu.sync_copy(srcr.at[tid], sidx)
    pltpu.sync_copy(dstr.at[tid], didx)
    _zero_vmem(zrow, ZROWS, FC)

    ncp = nch // 2
    for cg in range(nch):
        @pl.when(core == cg // ncp)
        def _(cg=cg):
            xs_c = ins[cg]
            out_c = outs[cg]
            for j in range(RPT // ZROWS):
                pltpu.sync_copy(zrow, acc.at[pl.ds(tid * RPT + j * ZROWS, ZROWS)])

            @pl.when(tid == NS - 1)
            def _():
                pltpu.sync_copy(zrow.at[pl.ds(0, 16)],
                                acc.at[pl.ds(NS * RPT, N_NODES - NS * RPT)])
            plsc.subcore_barrier()

            def g_start(b, j):
                pltpu.async_copy(xs_c.at[sidx.at[b]], bufs[j], gsems[j])

            def g_wait(b, j):
                pltpu.make_async_copy(xs_c.at[sidx.at[b]], bufs[j],
                                      gsems[j]).wait()

            def s_start(b, j):
                pltpu.async_copy(bufs[j], acc.at[didx.at[b]], ssems[j],
                                 add=True)

            def s_wait(b, j):
                pltpu.make_async_copy(bufs[j], acc.at[didx.at[b]],
                                      ssems[j]).wait()

            def slot(b, has_next):
                j = b % _NBUF
                jn = (b + _LOOK) % _NBUF
                g_wait(b, j)
                s_start(b, j)
                if has_next:
                    if b >= _LOOK:
                        s_wait(b - _LOOK, jn)
                    g_start(b + _LOOK, jn)

            for b in range(_LOOK):
                g_start(b, b % _NBUF)
            for b in range(_NBUF):
                slot(b, has_next=True)

            def group(k, _):
                base = _NBUF * k
                for j in range(_NBUF):
                    b = base + j
                    g_wait(b, j)
                    s_start(b, j)
                    s_wait(b - _LOOK, (j + _LOOK) % _NBUF)
                    g_start(b + _LOOK, (j + _LOOK) % _NBUF)
                return 0

            lax.fori_loop(1, (NB - _LOOK - 1) // _NBUF, group, 0)
            tail0 = ((NB - _LOOK - 1) // _NBUF) * _NBUF
            for b in range(tail0, NB):
                slot(b, has_next=(b + _LOOK < NB))
            for b in range(NB - _NBUF, NB):
                s_wait(b, b % _NBUF)
            plsc.subcore_barrier()
            pltpu.sync_copy(acc.at[pl.ds(tid * RPT, RPT)],
                            out_c.at[pl.ds(tid * RPT, RPT)])

            @pl.when(tid == NS - 1)
            def _():
                pltpu.sync_copy(acc.at[pl.ds(NS * RPT, N_NODES - NS * RPT)],
                                out_c.at[pl.ds(NS * RPT, N_NODES - NS * RPT)])


def _sc_aggregate(src_r, dst_r, chunks):
    nch = len(chunks)
    return pl.kernel(
        functools.partial(_agg_body, nch),
        mesh=_MESH(),
        compiler_params=pltpu.CompilerParams(use_tc_tiling_on_sc=False),
        out_type=[jax.ShapeDtypeStruct((N_NODES, FC), jnp.float32)] * nch,
        scratch_types=[
            pltpu.VMEM((NB, BATCH), jnp.int32),
            pltpu.VMEM((NB, BATCH), jnp.int32),
        ]
        + [pltpu.VMEM((BATCH, FC), jnp.float32)] * _NBUF
        + [
            pltpu.VMEM((ZROWS, FC), jnp.float32),
            pltpu.VMEM_SHARED((N_NODES, FC), jnp.float32),
        ]
        + [pltpu.SemaphoreType.DMA] * (2 * _NBUF),
    )(src_r, dst_r, *chunks)


_BM = 1000


def _dinv(deg_ref):
    return lax.rsqrt(deg_ref[...][:, 0:1] + 1.0)


def _prologue_kernel(x_ref, deg_ref, *out_refs):
    xs = x_ref[...] * _dinv(deg_ref)
    for c, o in enumerate(out_refs):
        o[...] = xs[:, c * FC:(c + 1) * FC]


def _layer_kernel(nch_in, *refs):
    agg = refs[:nch_in]
    xsp = refs[nch_in:2 * nch_in]
    deg_ref, w_ref, b_ref = refs[2 * nch_in:2 * nch_in + 3]
    out_refs = refs[2 * nch_in + 3:]
    di = _dinv(deg_ref)
    z = jnp.concatenate([a[...] + p[...] for a, p in zip(agg, xsp)], axis=1) * di
    h = jax.lax.dot_general(z, w_ref[...], (((1,), (0,)), ((), ())),
                            preferred_element_type=jnp.float32,
                            precision=lax.Precision.HIGHEST)
    h = jnp.maximum(h + b_ref[...][None, :], 0.0)
    xs = h * di
    for c, o in enumerate(out_refs):
        o[...] = xs[:, c * FC:(c + 1) * FC]


_NCH3 = 512 // FC


def _tail_kernel(*refs):
    agg = refs[:_NCH3]
    xsp = refs[_NCH3:2 * _NCH3]
    (deg_ref, w_ref, b_ref, batch_ref, wl_ref, bl_ref,
     out_ref, acc_s, cnt_s) = refs[2 * _NCH3:]
    i = pl.program_id(0)

    @pl.when(i == 0)
    def _():
        acc_s[...] = jnp.zeros_like(acc_s)
        cnt_s[...] = jnp.zeros_like(cnt_s)

    di = _dinv(deg_ref)
    z = jnp.concatenate([a[...] + p[...] for a, p in zip(agg, xsp)], axis=1) * di
    emb = jax.lax.dot_general(z, w_ref[...], (((1,), (0,)), ((), ())),
                              preferred_element_type=jnp.float32,
                              precision=lax.Precision.HIGHEST)
    emb = emb + b_ref[...][None, :]
    gids = lax.broadcasted_iota(jnp.int32, (_BM, N_GRAPHS), 1)
    s = (batch_ref[...] == gids).astype(jnp.float32)
    acc_s[...] += jax.lax.dot_general(s, emb, (((0,), (0,)), ((), ())),
                                      preferred_element_type=jnp.float32,
                                      precision=lax.Precision.HIGHEST)
    cnt_s[...] += jnp.sum(s, axis=0)[:, None]

    @pl.when(i == pl.num_programs(0) - 1)
    def _():
        pooled = acc_s[...] / jnp.maximum(cnt_s[...], 1.0)
        out_ref[...] = jax.lax.dot_general(
            pooled, wl_ref[...], (((1,), (0,)), ((), ())),
            preferred_element_type=jnp.float32,
            precision=lax.Precision.HIGHEST) + bl_ref[...][None, :]


def _row_spec(w):
    return pl.BlockSpec((_BM, w), lambda i: (i, 0))


def _full_spec(shape):
    nd = len(shape)
    return pl.BlockSpec(shape, lambda i: (0,) * nd)


def _tc_prologue(x, deg16):
    nch = x.shape[1] // FC
    return pl.pallas_call(
        _prologue_kernel,
        grid=(N_NODES // _BM,),
        in_specs=[_row_spec(x.shape[1]), _row_spec(16)],
        out_specs=[_row_spec(FC)] * nch,
        out_shape=[jax.ShapeDtypeStruct((N_NODES, FC), jnp.float32)] * nch,
    )(x, deg16)


def _tc_layer(agg, xsp, deg16, w, b):
    nch_in = len(agg)
    nch_out = w.shape[1] // FC
    return pl.pallas_call(
        functools.partial(_layer_kernel, nch_in),
        grid=(N_NODES // _BM,),
        in_specs=[_row_spec(FC)] * (2 * nch_in)
        + [_row_spec(16), _full_spec(w.shape), _full_spec(b.shape)],
        out_specs=[_row_spec(FC)] * nch_out,
        out_shape=[jax.ShapeDtypeStruct((N_NODES, FC), jnp.float32)] * nch_out,
    )(*agg, *xsp, deg16, w, b)


def _tc_tail(agg, xsp, deg16, w, b, batch2d, wl, bl):
    return pl.pallas_call(
        _tail_kernel,
        grid=(N_NODES // _BM,),
        in_specs=[_row_spec(FC)] * (2 * _NCH3)
        + [_row_spec(16), _full_spec(w.shape), _full_spec(b.shape),
           _row_spec(1), _full_spec(wl.shape), _full_spec(bl.shape)],
        out_specs=pl.BlockSpec((N_GRAPHS, wl.shape[1]), lambda i: (0, 0)),
        out_shape=jax.ShapeDtypeStruct((N_GRAPHS, wl.shape[1]), jnp.float32),
        scratch_shapes=[pltpu.VMEM((N_GRAPHS, w.shape[1]), jnp.float32),
                        pltpu.VMEM((N_GRAPHS, 1), jnp.float32)],
    )(*agg, *xsp, deg16, w, b, batch2d, wl, bl)


def kernel(x, edge_index, batch, W1, b1, W2, b2, W3, b3, Wl, bl):
    src_r = edge_index[0].astype(jnp.int32).reshape(NS, NB, BATCH)
    dst_r = edge_index[1].astype(jnp.int32).reshape(NS, NB, BATCH)
    batch2d = batch.astype(jnp.int32).reshape(N_NODES, 1)

    deg16 = _sc_degree(dst_r)

    _agg = _sc_aggregate
    xs0 = _tc_prologue(x, deg16)
    agg0 = _agg(src_r, dst_r, xs0)
    xs1 = _tc_layer(agg0, xs0, deg16, W1, b1)
    agg1 = _agg(src_r, dst_r, xs1)
    xs2 = _tc_layer(agg1, xs1, deg16, W2, b2)
    agg2 = _agg(src_r, dst_r, xs2)
    return _tc_tail(agg2, xs2, deg16, W3, b3, batch2d, Wl, bl)


# speedup vs baseline: 1.0242x; 1.0242x over previous
import functools

import jax
import jax.numpy as jnp
from jax import lax
from jax.experimental import pallas as pl
from jax.experimental.pallas import tpu as pltpu
from jax.experimental.pallas import tpu_sc as plsc

N_NODES = 10000
N_EDGES = 160000
N_GRAPHS = 64

NS = 16
FC = 64
EPT = N_EDGES // NS
BATCH = 80
NB = EPT // BATCH
ZROWS = 104
RPT = 624

_MESH = lambda: plsc.VectorSubcoreMesh(core_axis_name="c", subcore_axis_name="s")


def _zero_vmem(ref, nrows, ncols):
    zero16 = jnp.zeros((16,), jnp.float32)

    def zr(i, _):
        def zc(j, _):
            ref[i, pl.ds(j * 16, 16)] = zero16
            return 0
        return lax.fori_loop(0, ncols // 16, zc, 0)

    lax.fori_loop(0, nrows, zr, 0)


def _deg_body(dstr, out, didx, ones_v, zrow, acc):
    tid = lax.axis_index("s")
    core = lax.axis_index("c")

    @pl.when(core == 0)
    def _():
        pltpu.sync_copy(dstr.at[tid], didx)
        onerow = jnp.where(lax.iota(jnp.int32, 16) == 0, 1.0, 0.0).astype(jnp.float32)

        def fill(i, _):
            ones_v[i, :] = onerow
            return 0
        lax.fori_loop(0, BATCH, fill, 0)
        _zero_vmem(zrow, ZROWS, 16)
        for j in range(RPT // ZROWS):
            pltpu.sync_copy(zrow, acc.at[pl.ds(tid * RPT + j * ZROWS, ZROWS)])

        @pl.when(tid == NS - 1)
        def _():
            pltpu.sync_copy(zrow.at[pl.ds(0, 16)],
                            acc.at[pl.ds(NS * RPT, N_NODES - NS * RPT)])
        plsc.subcore_barrier()

        def body(b, _):
            pltpu.sync_copy(ones_v, acc.at[didx.at[b]], add=True)
            return 0
        lax.fori_loop(0, NB, body, 0)
        plsc.subcore_barrier()
        pltpu.sync_copy(acc.at[pl.ds(tid * RPT, RPT)],
                        out.at[pl.ds(tid * RPT, RPT)])

        @pl.when(tid == NS - 1)
        def _():
            pltpu.sync_copy(acc.at[pl.ds(NS * RPT, N_NODES - NS * RPT)],
                            out.at[pl.ds(NS * RPT, N_NODES - NS * RPT)])


def _sc_degree(dst_r):
    return pl.kernel(
        _deg_body,
        mesh=_MESH(),
        compiler_params=pltpu.CompilerParams(use_tc_tiling_on_sc=False),
        out_type=jax.ShapeDtypeStruct((N_NODES, 16), jnp.float32),
        scratch_types=[
            pltpu.VMEM((NB, BATCH), jnp.int32),
            pltpu.VMEM((BATCH, 16), jnp.float32),
            pltpu.VMEM((ZROWS, 16), jnp.float32),
            pltpu.VMEM_SHARED((N_NODES, 16), jnp.float32),
        ],
    )(dst_r)


_NBUF = 12
_LOOK = 6


def _agg_body(nch, srcr, dstr, *rest):
    ins = rest[:nch]
    outs = rest[nch:2 * nch]
    sidx = rest[2 * nch]
    didx = rest[2 * nch + 1]
    bufs = rest[2 * nch + 2:2 * nch + 2 + _NBUF]
    zrow = rest[2 * nch + 2 + _NBUF]
    acc = rest[2 * nch + 3 + _NBUF]
    gsems = rest[2 * nch + 4 + _NBUF:2 * nch + 4 + 2 * _NBUF]
    ssems = rest[2 * nch + 4 + 2 * _NBUF:]
    tid = lax.axis_index("s")
    core = lax.axis_index("c")

    pltpu.sync_copy(srcr.at[tid], sidx)
    pltpu.sync_copy(dstr.at[tid], didx)
    _zero_vmem(zrow, ZROWS, FC)

    ncp = nch // 2
    for cg in range(nch):
        @pl.when(core == cg // ncp)
        def _(cg=cg):
            xs_c = ins[cg]
            out_c = outs[cg]
            for j in range(RPT // ZROWS):
                pltpu.sync_copy(zrow, acc.at[pl.ds(tid * RPT + j * ZROWS, ZROWS)])

            @pl.when(tid == NS - 1)
            def _():
                pltpu.sync_copy(zrow.at[pl.ds(0, 16)],
                                acc.at[pl.ds(NS * RPT, N_NODES - NS * RPT)])
            plsc.subcore_barrier()

            def g_start(b, j):
                pltpu.async_copy(xs_c.at[sidx.at[b]], bufs[j], gsems[j])

            def g_wait(b, j):
                pltpu.make_async_copy(xs_c.at[sidx.at[b]], bufs[j],
                                      gsems[j]).wait()

            def s_start(b, j):
                pltpu.async_copy(bufs[j], acc.at[didx.at[b]], ssems[j],
                                 add=True)

            def s_wait(b, j):
                pltpu.make_async_copy(bufs[j], acc.at[didx.at[b]],
                                      ssems[j]).wait()

            def slot(b, has_next):
                j = b % _NBUF
                jn = (b + _LOOK) % _NBUF
                g_wait(b, j)
                s_start(b, j)
                if has_next:
                    if b >= _LOOK:
                        s_wait(b - _LOOK, jn)
                    g_start(b + _LOOK, jn)

            for b in range(_LOOK):
                g_start(b, b % _NBUF)
            for b in range(_NBUF):
                slot(b, has_next=True)

            def group(k, _):
                base = _NBUF * k
                for j in range(_NBUF):
                    b = base + j
                    g_wait(b, j)
                    s_start(b, j)
                    s_wait(b - _LOOK, (j + _LOOK) % _NBUF)
                    g_start(b + _LOOK, (j + _LOOK) % _NBUF)
                return 0

            lax.fori_loop(1, (NB - _LOOK - 1) // _NBUF, group, 0)
            tail0 = ((NB - _LOOK - 1) // _NBUF) * _NBUF
            for b in range(tail0, NB):
                slot(b, has_next=(b + _LOOK < NB))
            for b in range(NB - _NBUF, NB):
                s_wait(b, b % _NBUF)
            plsc.subcore_barrier()
            pltpu.sync_copy(acc.at[pl.ds(tid * RPT, RPT)],
                            out_c.at[pl.ds(tid * RPT, RPT)])

            @pl.when(tid == NS - 1)
            def _():
                pltpu.sync_copy(acc.at[pl.ds(NS * RPT, N_NODES - NS * RPT)],
                                out_c.at[pl.ds(NS * RPT, N_NODES - NS * RPT)])


def _sc_aggregate(src_r, dst_r, chunks):
    nch = len(chunks)
    return pl.kernel(
        functools.partial(_agg_body, nch),
        mesh=_MESH(),
        compiler_params=pltpu.CompilerParams(use_tc_tiling_on_sc=False),
        out_type=[jax.ShapeDtypeStruct((N_NODES, FC), jnp.float32)] * nch,
        scratch_types=[
            pltpu.VMEM((NB, BATCH), jnp.int32),
            pltpu.VMEM((NB, BATCH), jnp.int32),
        ]
        + [pltpu.VMEM((BATCH, FC), jnp.float32)] * _NBUF
        + [
            pltpu.VMEM((ZROWS, FC), jnp.float32),
            pltpu.VMEM_SHARED((N_NODES, FC), jnp.float32),
        ]
        + [pltpu.SemaphoreType.DMA] * (2 * _NBUF),
    )(src_r, dst_r, *chunks)


_BM = 1000


def _dinv(deg_ref):
    return lax.rsqrt(deg_ref[...][:, 0:1] + 1.0)


def _prologue_kernel(x_ref, deg_ref, *out_refs):
    xs = x_ref[...] * _dinv(deg_ref)
    for c, o in enumerate(out_refs):
        o[...] = xs[:, c * FC:(c + 1) * FC]


def _layer_kernel(nch_in, *refs):
    agg = refs[:nch_in]
    xsp = refs[nch_in:2 * nch_in]
    deg_ref, w_ref, b_ref = refs[2 * nch_in:2 * nch_in + 3]
    out_refs = refs[2 * nch_in + 3:]
    di = _dinv(deg_ref)
    z = jnp.concatenate([a[...] + p[...] for a, p in zip(agg, xsp)], axis=1) * di
    h = jax.lax.dot_general(z, w_ref[...], (((1,), (0,)), ((), ())),
                            preferred_element_type=jnp.float32,
                            precision=lax.Precision.HIGHEST)
    h = jnp.maximum(h + b_ref[...][None, :], 0.0)
    xs = h * di
    for c, o in enumerate(out_refs):
        o[...] = xs[:, c * FC:(c + 1) * FC]


_NCH3 = 512 // FC


def _tail_kernel(*refs):
    agg = refs[:_NCH3]
    xsp = refs[_NCH3:2 * _NCH3]
    (deg_ref, w_ref, b_ref, batch_ref, wl_ref, bl_ref,
     out_ref, acc_s, cnt_s) = refs[2 * _NCH3:]
    i = pl.program_id(0)

    @pl.when(i == 0)
    def _():
        acc_s[...] = jnp.zeros_like(acc_s)
        cnt_s[...] = jnp.zeros_like(cnt_s)

    di = _dinv(deg_ref)
    z = jnp.concatenate([a[...] + p[...] for a, p in zip(agg, xsp)], axis=1) * di
    emb = jax.lax.dot_general(z, w_ref[...], (((1,), (0,)), ((), ())),
                              preferred_element_type=jnp.float32,
                              precision=lax.Precision.HIGHEST)
    emb = emb + b_ref[...][None, :]
    gids = lax.broadcasted_iota(jnp.int32, (_BM, N_GRAPHS), 1)
    s = (batch_ref[...] == gids).astype(jnp.float32)
    acc_s[...] += jax.lax.dot_general(s, emb, (((0,), (0,)), ((), ())),
                                      preferred_element_type=jnp.float32,
                                      precision=lax.Precision.HIGHEST)
    cnt_s[...] += jnp.sum(s, axis=0)[:, None]

    @pl.when(i == pl.num_programs(0) - 1)
    def _():
        pooled = acc_s[...] / jnp.maximum(cnt_s[...], 1.0)
        out_ref[...] = jax.lax.dot_general(
            pooled, wl_ref[...], (((1,), (0,)), ((), ())),
            preferred_element_type=jnp.float32,
            precision=lax.Precision.HIGHEST) + bl_ref[...][None, :]


def _row_spec(w):
    return pl.BlockSpec((_BM, w), lambda i: (i, 0))


def _full_spec(shape):
    nd = len(shape)
    return pl.BlockSpec(shape, lambda i: (0,) * nd)


def _tc_prologue(x, deg16):
    nch = x.shape[1] // FC
    return pl.pallas_call(
        _prologue_kernel,
        grid=(N_NODES // _BM,),
        in_specs=[_row_spec(x.shape[1]), _row_spec(16)],
        out_specs=[_row_spec(FC)] * nch,
        out_shape=[jax.ShapeDtypeStruct((N_NODES, FC), jnp.float32)] * nch,
    )(x, deg16)


def _tc_layer(agg, xsp, deg16, w, b):
    nch_in = len(agg)
    nch_out = w.shape[1] // FC
    return pl.pallas_call(
        functools.partial(_layer_kernel, nch_in),
        grid=(N_NODES // _BM,),
        in_specs=[_row_spec(FC)] * (2 * nch_in)
        + [_row_spec(16), _full_spec(w.shape), _full_spec(b.shape)],
        out_specs=[_row_spec(FC)] * nch_out,
        out_shape=[jax.ShapeDtypeStruct((N_NODES, FC), jnp.float32)] * nch_out,
    )(*agg, *xsp, deg16, w, b)


def _tc_tail(agg, xsp, deg16, w, b, batch2d, wl, bl):
    return pl.pallas_call(
        _tail_kernel,
        grid=(N_NODES // _BM,),
        in_specs=[_row_spec(FC)] * (2 * _NCH3)
        + [_row_spec(16), _full_spec(w.shape), _full_spec(b.shape),
           _row_spec(1), _full_spec(wl.shape), _full_spec(bl.shape)],
        out_specs=pl.BlockSpec((N_GRAPHS, wl.shape[1]), lambda i: (0, 0)),
        out_shape=jax.ShapeDtypeStruct((N_GRAPHS, wl.shape[1]), jnp.float32),
        scratch_shapes=[pltpu.VMEM((N_GRAPHS, w.shape[1]), jnp.float32),
                        pltpu.VMEM((N_GRAPHS, 1), jnp.float32)],
    )(*agg, *xsp, deg16, w, b, batch2d, wl, bl)


def kernel(x, edge_index, batch, W1, b1, W2, b2, W3, b3, Wl, bl):
    src_r = edge_index[0].astype(jnp.int32).reshape(NS, NB, BATCH)
    dst_r = edge_index[1].astype(jnp.int32).reshape(NS, NB, BATCH)
    batch2d = batch.astype(jnp.int32).reshape(N_NODES, 1)

    deg16 = _sc_degree(dst_r)

    _agg = _sc_aggregate
    xs0 = _tc_prologue(x, deg16)
    agg0 = _agg(src_r, dst_r, xs0)
    xs1 = _tc_layer(agg0, xs0, deg16, W1, b1)
    agg1 = _agg(src_r, dst_r, xs1)
    xs2 = _tc_layer(agg1, xs1, deg16, W2, b2)
    agg2 = _agg(src_r, dst_r, xs2)
    return _tc_tail(agg2, xs2, deg16, W3, b3, batch2d, Wl, bl)
